# Initial kernel scaffold; baseline (speedup 1.0000x reference)
import jax
import jax.numpy as jnp
from jax.experimental import pallas as pl


def kernel(graph, feat, W1, al1, ar1, W2, al2, ar2, W3, al3, ar3, W4, al4, ar4):
    raise NotImplementedError("write your pallas kernel here")


# speedup vs baseline: 4.8996x; 4.8996x over previous
import jax
import jax.numpy as jnp
from jax import lax
from jax.experimental import pallas as pl
from jax.experimental.pallas import tpu as pltpu
from jax.experimental.pallas import tpu_sc as plsc

_N, _E, _D = 10000, 160000, 512
_L = 16
_S = 80
_R = 128
_NP = _S * _R
_NW = 32
_RPW = _R // _NW
_FB = 2048
_CAPH = 80 * _FB
_ECH = 8000
_LBLK = 2048
_GCH = 48
_BM = 640


def _wid():
    return lax.axis_index("s") * 2 + lax.axis_index("c")


def _mm_body(x_ref, w_ref, al_ref, ar_ref, h_ref, el_ref, er_ref):
    h = lax.dot_general(x_ref[...], w_ref[...], (((1,), (0,)), ((), ())),
                        preferred_element_type=jnp.float32)
    h_ref[...] = h
    el_ref[...] = jnp.sum(h * al_ref[...], axis=1, keepdims=True)
    er_ref[...] = jnp.sum(h * ar_ref[...], axis=1, keepdims=True)


def _matmul(x, w, al, ar):
    return pl.pallas_call(
        _mm_body,
        grid=(_NP // _BM,),
        in_specs=[pl.BlockSpec((_BM, _D), lambda i: (i, 0)),
                  pl.BlockSpec((_D, _D), lambda i: (0, 0)),
                  pl.BlockSpec((1, _D), lambda i: (0, 0)),
                  pl.BlockSpec((1, _D), lambda i: (0, 0))],
        out_specs=[pl.BlockSpec((_BM, _D), lambda i: (i, 0)),
                   pl.BlockSpec((_BM, 1), lambda i: (i, 0)),
                   pl.BlockSpec((_BM, 1), lambda i: (i, 0))],
        out_shape=[jax.ShapeDtypeStruct((_NP, _D), jnp.float32),
                   jax.ShapeDtypeStruct((_NP, 1), jnp.float32),
                   jax.ShapeDtypeStruct((_NP, 1), jnp.float32)],
    )(x, w, al, ar)


def _filter_body(src_hbm, dst_hbm, slist, llist, counts,
                 sbuf, dbuf, ls0, ls1, ls2, ls3, ll0, ll1, ll2, ll3, cstg):
    lsrc = [ls0, ls1, ls2, ls3]
    lldst = [ll0, ll1, ll2, ll3]
    w = _wid()
    io = lax.iota(jnp.int32, _L)

    def outer(c, carry):
        pltpu.sync_copy(src_hbm.at[pl.ds(pl.multiple_of(c * _ECH, 8), _ECH)], sbuf)
        pltpu.sync_copy(dst_hbm.at[pl.ds(pl.multiple_of(c * _ECH, 8), _ECH)], dbuf)

        def inner(g, car):
            sv = sbuf[pl.ds(g * _L, _L)]
            dv = dbuf[pl.ds(g * _L, _L)]
            new = []
            for rr in range(_RPW):
                wp, fl = car[2 * rr], car[2 * rr + 1]
                r = w * _RPW + rr
                lo = r * _S
                m = (dv >= lo) & (dv < lo + _S)
                ones = jnp.where(m, jnp.int32(1), jnp.int32(0))
                ranks = plsc.cumsum(ones)
                cnt = jnp.max(ranks)
                pos = wp + ranks - 1
                plsc.store_scatter(lsrc[rr], [pos], sv, mask=m)
                plsc.store_scatter(lldst[rr], [pos], dv - lo, mask=m)
                wp = wp + cnt
                full = wp >= _FB

                @pl.when(full)
                def _():
                    pltpu.sync_copy(lsrc[rr].at[pl.ds(0, _FB)],
                                    slist.at[pl.ds(pl.multiple_of(r * _CAPH + fl, 8), _FB)])
                    pltpu.sync_copy(lldst[rr].at[pl.ds(0, _FB)],
                                    llist.at[pl.ds(pl.multiple_of(r * _CAPH + fl, 8), _FB)])
                    t1 = lsrc[rr][pl.ds(_FB, _L)]
                    lsrc[rr][pl.ds(0, _L)] = t1
                    t2 = lldst[rr][pl.ds(_FB, _L)]
                    lldst[rr][pl.ds(0, _L)] = t2

                wp = jnp.where(full, wp - _FB, wp)
                fl = jnp.where(full, fl + _FB, fl)
                new += [wp, fl]
            return tuple(new)

        return lax.fori_loop(0, _ECH // _L, inner, carry)

    z = jnp.int32(0)
    carry = lax.fori_loop(0, _E // _ECH, outer, (z,) * (2 * _RPW))
    for rr in range(_RPW):
        wp, fl = carry[2 * rr], carry[2 * rr + 1]
        r = w * _RPW + rr
        pltpu.sync_copy(lsrc[rr].at[pl.ds(0, _FB)], slist.at[pl.ds(pl.multiple_of(r * _CAPH + fl, 8), _FB)])
        pltpu.sync_copy(lldst[rr].at[pl.ds(0, _FB)], llist.at[pl.ds(pl.multiple_of(r * _CAPH + fl, 8), _FB)])
        cstg[...] = jnp.zeros((_L,), jnp.int32) + (fl + wp)
        pltpu.sync_copy(cstg, counts.at[pl.ds(pl.multiple_of(r * _L, 8), _L)])


def _agg_body(h_hbm, el_hbm, er_hbm, slist, llist, counts, out_hbm,
              elv, accf, d2, erl, mpl, sblk, lblk, istg, gbuf, cstg,
              sem0, sem1):
    w = _wid()
    io = lax.iota(jnp.int32, _L)
    iof = io.astype(jnp.float32) * 0.0
    sems = [sem0, sem1]

    pltpu.sync_copy(el_hbm, elv)

    def mx(k, m):
        return jnp.maximum(m, elv[pl.ds(k * _L, _L)])
    mvec = lax.fori_loop(0, _NP // _L, mx,
                         jnp.full((_L,), -3.0e38, jnp.float32))
    emax = jnp.max(mvec)

    for rr in range(_RPW):
        r = w * _RPW + rr
        lo = r * _S
        pltpu.sync_copy(er_hbm.at[pl.ds(pl.multiple_of(lo, 8), _S)], erl)
        for k in range(_S // _L):
            xv = emax + erl[pl.ds(k * _L, _L)]
            mpl[pl.ds(k * _L, _L)] = jnp.where(xv > 0, xv, 0.2 * xv)

        def zacc(k, _):
            accf[pl.ds(k * _L, _L)] = jnp.zeros((_L,), jnp.float32)
            return 0
        lax.fori_loop(0, (_S * _D) // _L, zacc, 0)

        def zd2(k, _):
            d2[pl.ds(k * _L, _L)] = jnp.zeros((_L,), jnp.float32)
            return 0
        lax.fori_loop(0, (_S * _L) // _L, zd2, 0)

        pltpu.sync_copy(counts.at[pl.ds(pl.multiple_of(r * _L, 8), _L)], cstg)
        n_e = jnp.max(cstg[...])
        n_b = (n_e + _LBLK - 1) // _LBLK

        def block(b, _):
            pltpu.sync_copy(slist.at[pl.ds(pl.multiple_of(r * _CAPH + b * _LBLK, 8), _LBLK)], sblk)
            pltpu.sync_copy(llist.at[pl.ds(pl.multiple_of(r * _CAPH + b * _LBLK, 8), _LBLK)], lblk)
            n_in = jnp.minimum(jnp.int32(_LBLK), n_e - b * _LBLK)
            n_g = (n_in + _GCH - 1) // _GCH

            def fire(c, bb):
                for v in range(_GCH // _L):
                    off = c * _GCH + v * _L
                    sv = sblk[pl.ds(off, _L)]
                    ok = (io + off) < n_in
                    istg[bb, pl.ds(v * _L, _L)] = jnp.where(ok, sv, 0)
                pltpu.make_async_copy(h_hbm.at[istg.at[bb]], gbuf.at[bb],
                                      sems[bb]).start()

            def wait(bb):
                pltpu.make_async_copy(h_hbm.at[istg.at[bb]], gbuf.at[bb],
                                      sems[bb]).wait()

            def process(c, bb):
                def grp(gi, _g):
                    off = c * _GCH + gi * _L
                    ok = (io + off) < n_in
                    lc = jnp.where(ok, lblk[pl.ds(off, _L)], 0)
                    sc_ = jnp.where(ok, sblk[pl.ds(off, _L)], 0)
                    av = plsc.load_gather(elv, [sc_])
                    bv = plsc.load_gather(erl, [lc])
                    mv = plsc.load_gather(mpl, [lc])
                    xv = av + bv
                    ev = jnp.where(xv > 0, xv, 0.2 * xv)
                    ex = jnp.exp(ev - mv)
                    ex = jnp.where(ok, ex, 0.0)

                    def edge(j, _e):
                        sel = io == j
                        exj = jnp.sum(jnp.where(sel, ex, 0.0))
                        ldj = jnp.sum(jnp.where(sel, lc, 0))
                        base = ldj * _D
                        row = gi * _L + j
                        for k in range(_D // _L):
                            gv = gbuf[bb, row, pl.ds(k * _L, _L)]
                            plsc.addupdate(accf.at[pl.ds(pl.multiple_of(base + k * _L, 8), _L)],
                                           exj * gv)
                        plsc.addupdate(d2.at[pl.ds(pl.multiple_of(ldj * _L, 8), _L)], iof + exj)
                        return 0
                    lax.fori_loop(0, _L, edge, 0)
                    return 0
                lax.fori_loop(0, _GCH // _L, grp, 0)

            for bb0 in range(2):
                @pl.when(bb0 < n_g)
                def _():
                    fire(jnp.int32(bb0), bb0)

            def pair(p, _p):
                for bb in range(2):
                    c = 2 * p + bb

                    @pl.when(c < n_g)
                    def _():
                        wait(bb)
                        process(c, bb)

                        @pl.when(c + 2 < n_g)
                        def _():
                            fire(c + 2, bb)
                return 0
            lax.fori_loop(0, (n_g + 1) // 2, pair, 0)
            return 0
        lax.fori_loop(0, n_b, block, 0)

        def fin(g, _):
            idxv = (io + g * _L) * _L
            dv = plsc.load_gather(d2, [idxv])
            rec = 1.0 / (dv + 1e-9)

            def node(j, _2):
                nn = g * _L + j
                rb = jnp.sum(jnp.where(io == j, rec, 0.0))
                base = nn * _D
                for k in range(_D // _L):
                    v = accf[pl.ds(pl.multiple_of(base + k * _L, 8), _L)] * rb
                    o = jnp.where(v > 0, v, jnp.exp(jnp.minimum(v, 0.0)) - 1.0)
                    accf[pl.ds(pl.multiple_of(base + k * _L, 8), _L)] = o
                return 0
            lax.fori_loop(0, _L, node, 0)
            return 0
        lax.fori_loop(0, _S // _L, fin, 0)
        pltpu.sync_copy(accf, out_hbm.at[pl.ds(pl.multiple_of(lo * _D, 8), _S * _D)])


_sc_kernels_cache = []


def _sc_kernels():
    if not _sc_kernels_cache:
        mesh = plsc.VectorSubcoreMesh(core_axis_name="c", subcore_axis_name="s")
        cp = pltpu.CompilerParams(needs_layout_passes=False)
        filt = pl.kernel(
            _filter_body,
            mesh=mesh,
            compiler_params=cp,
            out_type=[jax.ShapeDtypeStruct((_R * _CAPH,), jnp.int32),
                      jax.ShapeDtypeStruct((_R * _CAPH,), jnp.int32),
                      jax.ShapeDtypeStruct((_R * _L,), jnp.int32)],
            scratch_types=[pltpu.VMEM((_ECH,), jnp.int32),
                           pltpu.VMEM((_ECH,), jnp.int32),
                           pltpu.VMEM((_FB + 128,), jnp.int32),
                           pltpu.VMEM((_FB + 128,), jnp.int32),
                           pltpu.VMEM((_FB + 128,), jnp.int32),
                           pltpu.VMEM((_FB + 128,), jnp.int32),
                           pltpu.VMEM((_FB + 128,), jnp.int32),
                           pltpu.VMEM((_FB + 128,), jnp.int32),
                           pltpu.VMEM((_FB + 128,), jnp.int32),
                           pltpu.VMEM((_FB + 128,), jnp.int32),
                           pltpu.VMEM((_L,), jnp.int32)],
        )
        agg = pl.kernel(
            _agg_body,
            mesh=mesh,
            compiler_params=cp,
            out_type=[jax.ShapeDtypeStruct((_NP * _D,), jnp.float32)],
            scratch_types=[pltpu.VMEM((_NP,), jnp.float32),
                           pltpu.VMEM((_S * _D,), jnp.float32),
                           pltpu.VMEM((_S * _L,), jnp.float32),
                           pltpu.VMEM((_S,), jnp.float32),
                           pltpu.VMEM((_S,), jnp.float32),
                           pltpu.VMEM((_LBLK,), jnp.int32),
                           pltpu.VMEM((_LBLK,), jnp.int32),
                           pltpu.VMEM((2, _GCH), jnp.int32),
                           pltpu.VMEM((2, _GCH, _D), jnp.float32),
                           pltpu.VMEM((_L,), jnp.int32),
                           pltpu.SemaphoreType.DMA,
                           pltpu.SemaphoreType.DMA],
        )
        _sc_kernels_cache.append((filt, agg))
    return _sc_kernels_cache[0]


def kernel(graph, feat, W1, al1, ar1, W2, al2, ar2, W3, al3, ar3,
           W4, al4, ar4):
    src = graph[0]
    dst = graph[1]
    x = jnp.zeros((_NP, _D), jnp.float32).at[:_N].set(feat)
    _filter, _aggregate = _sc_kernels()
    slist, llist, counts = _filter(src, dst)
    for wmat, al, ar in ((W1, al1, ar1), (W2, al2, ar2),
                         (W3, al3, ar3), (W4, al4, ar4)):
        h, el, er = _matmul(x, wmat, al.reshape(1, _D), ar.reshape(1, _D))
        (outf,) = _aggregate(h, el.reshape(_NP), er.reshape(_NP),
                             slist, llist, counts)
        x = outf.reshape(_NP, _D)
    return x[:_N]
